# baseline (speedup 1.0000x reference)
import functools

import jax
import jax.numpy as jnp
from jax import lax
from jax.experimental import pallas as pl
from jax.experimental.pallas import tpu as pltpu
from jax.experimental.pallas import tpu_sc as plsc

_NC = 2
_NS = 16
_NW = _NC * _NS

_TR = 32
_CW = _TR * 1024


def _sc_body(B, L, D, x_hbm, pe_hbm, o_hbm, pe_buf, xb0, xb1,
             si0, si1, so0, so1):
    w = lax.axis_index("s") * _NC + lax.axis_index("c")
    rows_per_w = L // _NW
    pe_off = w * rows_per_w * D
    n_chunks = rows_per_w * D // _CW
    nsteps = n_chunks * B
    xbufs = (xb0, xb1)
    sin = (si0, si1)
    sout = (so0, so1)

    def xoff(t):
        c, b = divmod(t, B)
        return b * L * D + pe_off + c * _CW

    pltpu.async_copy(x_hbm.at[pl.ds(xoff(0), _CW)], xbufs[0], sin[0])
    for t in range(nsteps):
        c, b = divmod(t, B)
        cur = t % 2
        if b == 0:
            pltpu.sync_copy(pe_hbm.at[pl.ds(pe_off + c * _CW, _CW)], pe_buf)
        pltpu.make_async_copy(
            x_hbm.at[pl.ds(xoff(t), _CW)], xbufs[cur], sin[cur]).wait()
        if t + 1 < nsteps:
            if t >= 1:
                pltpu.make_async_copy(
                    xbufs[1 - cur], o_hbm.at[pl.ds(xoff(t - 1), _CW)],
                    sout[1 - cur]).wait()
            pltpu.async_copy(
                x_hbm.at[pl.ds(xoff(t + 1), _CW)], xbufs[1 - cur],
                sin[1 - cur])

        xb = xbufs[cur]

        @plsc.parallel_loop(0, _CW, 16, unroll=8)
        def _vec_add(i):
            s = pl.ds(i, 16)
            plsc.addupdate(xb.at[s], pe_buf[s])

        pltpu.async_copy(xb, o_hbm.at[pl.ds(xoff(t), _CW)], sout[cur])

    for t in (nsteps - 2, nsteps - 1):
        pltpu.make_async_copy(
            xbufs[t % 2], o_hbm.at[pl.ds(xoff(t), _CW)], sout[t % 2]).wait()


def kernel(x, pe):
    B, L, D = x.shape
    xf = x.reshape(B * L * D)
    pef = pe.reshape(pe.shape[0] * D)
    mesh = plsc.VectorSubcoreMesh(core_axis_name="c", subcore_axis_name="s")
    sc_call = functools.partial(
        pl.kernel,
        mesh=mesh,
        out_type=jax.ShapeDtypeStruct((B * L * D,), x.dtype),
        scratch_types=[
            pltpu.VMEM((_CW,), jnp.float32),
            pltpu.VMEM((_CW,), jnp.float32),
            pltpu.VMEM((_CW,), jnp.float32),
            pltpu.SemaphoreType.DMA,
            pltpu.SemaphoreType.DMA,
            pltpu.SemaphoreType.DMA,
            pltpu.SemaphoreType.DMA,
        ],
    )(functools.partial(_sc_body, B, L, D))
    out = sc_call(xf, pef)
    return out.reshape(B, L, D)


# speedup vs baseline: 2.5865x; 2.5865x over previous
import functools

import jax
import jax.numpy as jnp
from jax import lax
from jax.experimental import pallas as pl
from jax.experimental.pallas import tpu as pltpu
from jax.experimental.pallas import tpu_sc as plsc

_NC = 2
_NS = 16
_NW = _NC * _NS

_TR = 32
_CW = _TR * 1024


def _sc_body(B, L, D, x_hbm, pe_hbm, o_hbm, pe_buf, xb0, xb1,
             si0, si1, so0, so1):
    w = lax.axis_index("s") * _NC + lax.axis_index("c")
    rows_per_w = L // _NW
    pe_row0 = w * rows_per_w
    n_chunks = rows_per_w // _TR
    nsteps = n_chunks * B
    xbufs = (xb0, xb1)
    sin = (si0, si1)
    sout = (so0, so1)

    def xrow(t):
        c, b = divmod(t, B)
        return b * L + pe_row0 + c * _TR

    pltpu.async_copy(x_hbm.at[pl.ds(xrow(0), _TR)], xbufs[0], sin[0])
    for t in range(nsteps):
        c, b = divmod(t, B)
        cur = t % 2
        if b == 0:
            pltpu.sync_copy(pe_hbm.at[pl.ds(pe_row0 + c * _TR, _TR)], pe_buf)
        pltpu.make_async_copy(
            x_hbm.at[pl.ds(xrow(t), _TR)], xbufs[cur], sin[cur]).wait()
        if t + 1 < nsteps:
            if t >= 1:
                pltpu.make_async_copy(
                    xbufs[1 - cur], o_hbm.at[pl.ds(xrow(t - 1), _TR)],
                    sout[1 - cur]).wait()
            pltpu.async_copy(
                x_hbm.at[pl.ds(xrow(t + 1), _TR)], xbufs[1 - cur],
                sin[1 - cur])

        xb = xbufs[cur]

        def row_add(r, _):
            @plsc.parallel_loop(0, D, 16, unroll=8)
            def _vec_add(i):
                s = pl.ds(i, 16)
                plsc.addupdate(xb.at[r, s], pe_buf[r, s])
            return 0

        lax.fori_loop(0, _TR, row_add, 0)
        pltpu.async_copy(xb, o_hbm.at[pl.ds(xrow(t), _TR)], sout[cur])

    for t in (nsteps - 2, nsteps - 1):
        pltpu.make_async_copy(
            xbufs[t % 2], o_hbm.at[pl.ds(xrow(t), _TR)], sout[t % 2]).wait()


def kernel(x, pe):
    B, L, D = x.shape
    xf = x.reshape(B * L, D)
    mesh = plsc.VectorSubcoreMesh(core_axis_name="c", subcore_axis_name="s")
    sc_call = functools.partial(
        pl.kernel,
        mesh=mesh,
        out_type=jax.ShapeDtypeStruct((B * L, D), x.dtype),
        compiler_params=pltpu.CompilerParams(use_tc_tiling_on_sc=True),
        scratch_types=[
            pltpu.VMEM((_TR, D), jnp.float32),
            pltpu.VMEM((_TR, D), jnp.float32),
            pltpu.VMEM((_TR, D), jnp.float32),
            pltpu.SemaphoreType.DMA,
            pltpu.SemaphoreType.DMA,
            pltpu.SemaphoreType.DMA,
            pltpu.SemaphoreType.DMA,
        ],
    )(functools.partial(_sc_body, B, L, D))
    out = sc_call(xf, pe)
    return out.reshape(B, L, D)


# speedup vs baseline: 2.8116x; 1.0870x over previous
import functools

import jax
import jax.numpy as jnp
from jax import lax
from jax.experimental import pallas as pl
from jax.experimental.pallas import tpu as pltpu
from jax.experimental.pallas import tpu_sc as plsc

_NC = 2
_NS = 16
_NW = _NC * _NS

_TR = 16
_NBUF = 4


def _sc_body(B, L, D, x_hbm, pe_hbm, o_hbm, pe_buf, xbufs, sin, sout):
    w = lax.axis_index("s") * _NC + lax.axis_index("c")
    rows_per_w = L // _NW
    pe_row0 = w * rows_per_w
    n_chunks = rows_per_w // _TR
    nsteps = n_chunks * B

    def xrow(t):
        c, b = divmod(t, B)
        return b * L + pe_row0 + c * _TR

    def start_in(t):
        pltpu.async_copy(
            x_hbm.at[pl.ds(xrow(t), _TR)], xbufs[t % _NBUF], sin[t % _NBUF])

    def wait_in(t):
        pltpu.make_async_copy(
            x_hbm.at[pl.ds(xrow(t), _TR)], xbufs[t % _NBUF],
            sin[t % _NBUF]).wait()

    def start_out(t):
        pltpu.async_copy(
            xbufs[t % _NBUF], o_hbm.at[pl.ds(xrow(t), _TR)], sout[t % _NBUF])

    def wait_out(t):
        pltpu.make_async_copy(
            xbufs[t % _NBUF], o_hbm.at[pl.ds(xrow(t), _TR)],
            sout[t % _NBUF]).wait()

    start_in(0)
    start_in(1)
    for t in range(nsteps):
        c, b = divmod(t, B)
        if b == 0:
            pltpu.sync_copy(pe_hbm.at[pl.ds(pe_row0 + c * _TR, _TR)], pe_buf)
        wait_in(t)
        if t + 2 < nsteps:
            if t - 2 >= 0:
                wait_out(t - 2)
            start_in(t + 2)

        xb = xbufs[t % _NBUF]

        def row_add(r, _):
            @plsc.parallel_loop(0, D, 16, unroll=8)
            def _vec_add(i):
                s = pl.ds(i, 16)
                plsc.addupdate(xb.at[r, s], pe_buf[r, s])
            return 0

        lax.fori_loop(0, _TR, row_add, 0)
        start_out(t)

    for t in (nsteps - 2, nsteps - 1):
        wait_out(t)


def kernel(x, pe):
    B, L, D = x.shape
    xf = x.reshape(B * L, D)
    mesh = plsc.VectorSubcoreMesh(core_axis_name="c", subcore_axis_name="s")

    def body(x_hbm, pe_hbm, o_hbm, pe_buf, xb0, xb1, xb2, xb3,
             si0, si1, si2, si3, so0, so1, so2, so3):
        _sc_body(B, L, D, x_hbm, pe_hbm, o_hbm, pe_buf,
                 (xb0, xb1, xb2, xb3), (si0, si1, si2, si3),
                 (so0, so1, so2, so3))

    sc_call = functools.partial(
        pl.kernel,
        mesh=mesh,
        out_type=jax.ShapeDtypeStruct((B * L, D), x.dtype),
        compiler_params=pltpu.CompilerParams(use_tc_tiling_on_sc=True),
        scratch_types=(
            [pltpu.VMEM((_TR, D), jnp.float32)] * (1 + _NBUF)
            + [pltpu.SemaphoreType.DMA] * (2 * _NBUF)
        ),
    )(body)
    out = sc_call(xf, pe)
    return out.reshape(B, L, D)


# speedup vs baseline: 2.9670x; 1.0553x over previous
import functools

import jax
import jax.numpy as jnp
from jax import lax
from jax.experimental import pallas as pl
from jax.experimental.pallas import tpu as pltpu
from jax.experimental.pallas import tpu_sc as plsc

_NC = 2
_NS = 16
_NW = _NC * _NS

_TR = 16
_NBUF = 6
_LOOKAHEAD = _NBUF - 2


def _sc_body(B, L, D, x_hbm, pe_hbm, o_hbm, pe_bufs, xbufs, spe, sin, sout):
    w = lax.axis_index("s") * _NC + lax.axis_index("c")
    rows_per_w = L // _NW
    pe_row0 = w * rows_per_w
    n_chunks = rows_per_w // _TR
    nsteps = n_chunks * B

    def xrow(t):
        c, b = divmod(t, B)
        return b * L + pe_row0 + c * _TR

    def start_in(t):
        pltpu.async_copy(
            x_hbm.at[pl.ds(xrow(t), _TR)], xbufs[t % _NBUF], sin[t % _NBUF])

    def wait_in(t):
        pltpu.make_async_copy(
            x_hbm.at[pl.ds(xrow(t), _TR)], xbufs[t % _NBUF],
            sin[t % _NBUF]).wait()

    def start_out(t):
        pltpu.async_copy(
            xbufs[t % _NBUF], o_hbm.at[pl.ds(xrow(t), _TR)], sout[t % _NBUF])

    def wait_out(t):
        pltpu.make_async_copy(
            xbufs[t % _NBUF], o_hbm.at[pl.ds(xrow(t), _TR)],
            sout[t % _NBUF]).wait()

    def start_pe(c):
        pltpu.async_copy(
            pe_hbm.at[pl.ds(pe_row0 + c * _TR, _TR)], pe_bufs[c % 2],
            spe[c % 2])

    def wait_pe(c):
        pltpu.make_async_copy(
            pe_hbm.at[pl.ds(pe_row0 + c * _TR, _TR)], pe_bufs[c % 2],
            spe[c % 2]).wait()

    start_pe(0)
    for t in range(_LOOKAHEAD):
        start_in(t)
    for t in range(nsteps):
        c, b = divmod(t, B)
        if b == 0:
            wait_pe(c)
            if c + 1 < n_chunks:
                start_pe(c + 1)
        wait_in(t)
        if t + _LOOKAHEAD < nsteps:
            if t - 2 >= 0:
                wait_out(t - 2)
            start_in(t + _LOOKAHEAD)

        xb = xbufs[t % _NBUF]
        pe_buf = pe_bufs[c % 2]

        def row_add(r, _):
            @plsc.parallel_loop(0, D, 16, unroll=16)
            def _vec_add(i):
                s = pl.ds(i, 16)
                plsc.addupdate(xb.at[r, s], pe_buf[r, s])
            return 0

        lax.fori_loop(0, _TR, row_add, 0)
        start_out(t)

    for t in range(max(0, nsteps - 2), nsteps):
        wait_out(t)


def kernel(x, pe):
    B, L, D = x.shape
    xf = x.reshape(B * L, D)
    mesh = plsc.VectorSubcoreMesh(core_axis_name="c", subcore_axis_name="s")

    def body(x_hbm, pe_hbm, o_hbm, *scratch):
        pe_bufs = scratch[0:2]
        xbufs = scratch[2:2 + _NBUF]
        spe = scratch[2 + _NBUF:4 + _NBUF]
        sin = scratch[4 + _NBUF:4 + 2 * _NBUF]
        sout = scratch[4 + 2 * _NBUF:4 + 3 * _NBUF]
        _sc_body(B, L, D, x_hbm, pe_hbm, o_hbm, pe_bufs, xbufs, spe, sin,
                 sout)

    sc_call = functools.partial(
        pl.kernel,
        mesh=mesh,
        out_type=jax.ShapeDtypeStruct((B * L, D), x.dtype),
        compiler_params=pltpu.CompilerParams(use_tc_tiling_on_sc=True),
        scratch_types=(
            [pltpu.VMEM((_TR, D), jnp.float32)] * (2 + _NBUF)
            + [pltpu.SemaphoreType.DMA] * (2 + 2 * _NBUF)
        ),
    )(body)
    out = sc_call(xf, pe)
    return out.reshape(B, L, D)
